# baseline (speedup 1.0000x reference)
import jax
import jax.numpy as jnp
from jax.experimental import pallas as pl
from jax.experimental.pallas import tpu as pltpu

_B = 8
_TOTAL = 16384
_D = 4096
_CHUNK = 256
_NCHUNK = _TOTAL // _CHUNK
_NBUF = 12


def _copy_cu_kernel(len_ref, in_ref, out_ref, cu_ref, buf, rsem, wsem):
    cu_ref[0] = jnp.int32(0)
    acc = jnp.int32(0)
    for i in range(_B):
        acc = acc + len_ref[i]
        cu_ref[i + 1] = acc

    def read(c):
        b = c % _NBUF
        pltpu.make_async_copy(
            in_ref.at[pl.ds(c * _CHUNK, _CHUNK), :], buf.at[b], rsem.at[b]
        ).start()

    def write(c):
        b = c % _NBUF
        pltpu.make_async_copy(
            buf.at[b], out_ref.at[pl.ds(c * _CHUNK, _CHUNK), :], wsem.at[b]
        ).start()

    for c in range(_NBUF):
        read(c)
    for c in range(_NCHUNK):
        b = c % _NBUF
        pltpu.make_async_copy(
            in_ref.at[pl.ds(c * _CHUNK, _CHUNK), :], buf.at[b], rsem.at[b]
        ).wait()
        write(c)
        nc = c + _NBUF
        if nc < _NCHUNK:
            pltpu.make_async_copy(
                buf.at[b], out_ref.at[pl.ds(c * _CHUNK, _CHUNK), :], wsem.at[b]
            ).wait()
            read(nc)
    for c in range(_NCHUNK - _NBUF, _NCHUNK):
        b = c % _NBUF
        pltpu.make_async_copy(
            buf.at[b], out_ref.at[pl.ds(c * _CHUNK, _CHUNK), :], wsem.at[b]
        ).wait()


def kernel(hidden_states, lengths_cpu):
    lengths = lengths_cpu.astype(jnp.int32)
    values, cu_lengths = pl.pallas_call(
        _copy_cu_kernel,
        in_specs=[
            pl.BlockSpec(memory_space=pltpu.SMEM),
            pl.BlockSpec(memory_space=pl.ANY),
        ],
        out_specs=[
            pl.BlockSpec(memory_space=pl.ANY),
            pl.BlockSpec(memory_space=pltpu.SMEM),
        ],
        out_shape=[
            jax.ShapeDtypeStruct((_TOTAL, _D), jnp.float32),
            jax.ShapeDtypeStruct((_B + 1,), jnp.int32),
        ],
        scratch_shapes=[
            pltpu.VMEM((_NBUF, _CHUNK, _D), jnp.float32),
            pltpu.SemaphoreType.DMA((_NBUF,)),
            pltpu.SemaphoreType.DMA((_NBUF,)),
        ],
    )(lengths, hidden_states)
    return values, cu_lengths


# speedup vs baseline: 1.0103x; 1.0103x over previous
import jax
import jax.numpy as jnp
from jax.experimental import pallas as pl
from jax.experimental.pallas import tpu as pltpu

_B = 8
_TOTAL = 16384
_D = 4096
_CHUNK = 512
_NCHUNK = _TOTAL // _CHUNK
_NBUF = 7


def _copy_cu_kernel(len_ref, in_ref, out_ref, cu_ref, buf, rsem, wsem):
    cu_ref[0] = jnp.int32(0)
    acc = jnp.int32(0)
    for i in range(_B):
        acc = acc + len_ref[i]
        cu_ref[i + 1] = acc

    def read(c):
        b = c % _NBUF
        pltpu.make_async_copy(
            in_ref.at[pl.ds(c * _CHUNK, _CHUNK), :], buf.at[b], rsem.at[b]
        ).start()

    def write(c):
        b = c % _NBUF
        pltpu.make_async_copy(
            buf.at[b], out_ref.at[pl.ds(c * _CHUNK, _CHUNK), :], wsem.at[b]
        ).start()

    for c in range(_NBUF):
        read(c)
    for c in range(_NCHUNK):
        b = c % _NBUF
        pltpu.make_async_copy(
            in_ref.at[pl.ds(c * _CHUNK, _CHUNK), :], buf.at[b], rsem.at[b]
        ).wait()
        write(c)
        nc = c + _NBUF
        if nc < _NCHUNK:
            pltpu.make_async_copy(
                buf.at[b], out_ref.at[pl.ds(c * _CHUNK, _CHUNK), :], wsem.at[b]
            ).wait()
            read(nc)
    for c in range(_NCHUNK - _NBUF, _NCHUNK):
        b = c % _NBUF
        pltpu.make_async_copy(
            buf.at[b], out_ref.at[pl.ds(c * _CHUNK, _CHUNK), :], wsem.at[b]
        ).wait()


def kernel(hidden_states, lengths_cpu):
    lengths = lengths_cpu.astype(jnp.int32)
    values, cu_lengths = pl.pallas_call(
        _copy_cu_kernel,
        in_specs=[
            pl.BlockSpec(memory_space=pltpu.SMEM),
            pl.BlockSpec(memory_space=pl.ANY),
        ],
        out_specs=[
            pl.BlockSpec(memory_space=pl.ANY),
            pl.BlockSpec(memory_space=pltpu.SMEM),
        ],
        out_shape=[
            jax.ShapeDtypeStruct((_TOTAL, _D), jnp.float32),
            jax.ShapeDtypeStruct((_B + 1,), jnp.int32),
        ],
        scratch_shapes=[
            pltpu.VMEM((_NBUF, _CHUNK, _D), jnp.float32),
            pltpu.SemaphoreType.DMA((_NBUF,)),
            pltpu.SemaphoreType.DMA((_NBUF,)),
        ],
    )(lengths, hidden_states)
    return values, cu_lengths
